# baseline (speedup 1.0000x reference)
import dataclasses
import functools

import jax
import jax.numpy as jnp
from jax import lax
from jax.experimental import pallas as pl
from jax.experimental.pallas import tpu as pltpu
from jax.experimental.pallas import tpu_sc as plsc

NC = 2
NS = 16
NW = NC * NS
CHUNK = 128
KGROUP = 2
ROW_BLOCK = 2000


def _sc_segment_sum(t, idx3, n_pad, with_cnt):
    n_chunks = idx3.shape[0] // (2 * NW)
    d = t.shape[1]
    rows_per_sub = n_pad // NS
    n_groups = n_chunks // KGROUP
    n_lin = rows_per_sub // CHUNK

    out_type = [jax.ShapeDtypeStruct((NC * n_pad, d), jnp.float32)]
    scratch = [
        pltpu.VMEM((2 * KGROUP, CHUNK), jnp.int32),
        pltpu.VMEM((n_lin, CHUNK), jnp.int32),
        pltpu.VMEM((KGROUP, CHUNK, d), jnp.float32),
        pltpu.SemaphoreType.DMA,
        pltpu.SemaphoreType.DMA,
        pltpu.VMEM_SHARED((n_pad, d), jnp.float32),
    ]
    if with_cnt:
        out_type.append(jax.ShapeDtypeStruct((NW, n_pad), jnp.float32))
        scratch.append(pltpu.VMEM((n_pad,), jnp.float32))

    mesh = plsc.VectorSubcoreMesh(core_axis_name="c", subcore_axis_name="s",
                                  num_cores=NC)
    cp = pltpu.CompilerParams()
    if "needs_layout_passes" in pltpu.CompilerParams.__dataclass_fields__:
        cp = dataclasses.replace(cp, needs_layout_passes=False)

    @functools.partial(pl.kernel, out_type=out_type, mesh=mesh,
                       scratch_types=scratch, compiler_params=cp)
    def k(t_hbm, idx_hbm, lin_hbm, *rest):
        if with_cnt:
            (agg_hbm, cnt_hbm, idx_v, lin_v, rows_v, gsem, ssem,
             agg_sh, cnt_tile) = rest
        else:
            agg_hbm, idx_v, lin_v, rows_v, gsem, ssem, agg_sh = rest
        sem = gsem
        cid = lax.axis_index("c")
        sid = lax.axis_index("s")
        wid = cid * NS + sid
        base = sid * rows_per_sub
        idx_base = wid * 2 * n_chunks

        pltpu.sync_copy(lin_hbm.at[sid], lin_v)

        @pl.loop(0, CHUNK)
        def _(i):
            @pl.loop(0, d, step=16)
            def _(j):
                rows_v[0, i, pl.ds(j, 16)] = jnp.zeros((16,), jnp.float32)

        if with_cnt:
            @pl.loop(0, n_pad, step=16)
            def _(i):
                cnt_tile[pl.ds(i, 16)] = jnp.zeros((16,), jnp.float32)

        for r in range(n_lin):
            pltpu.sync_copy(rows_v.at[0], agg_sh.at[lin_v.at[r]])

        plsc.subcore_barrier()

        @pl.loop(0, n_groups)
        def _(g):
            pltpu.sync_copy(
                idx_hbm.at[pl.ds(idx_base + g * 2 * KGROUP, 2 * KGROUP)],
                idx_v)
            gh = [pltpu.async_copy(t_hbm.at[idx_v.at[2 * kk]],
                                   rows_v.at[kk], gsem)
                  for kk in range(KGROUP)]
            for h in gh:
                h.wait()
            sh = [pltpu.async_copy(rows_v.at[kk],
                                   agg_sh.at[idx_v.at[2 * kk + 1]], ssem,
                                   add=True)
                  for kk in range(KGROUP)]
            if with_cnt:
                for kk in range(KGROUP):
                    for l in range(CHUNK // 16):
                        dst16 = idx_v[2 * kk + 1, pl.ds(l * 16, 16)]
                        plsc.addupdate_scatter(
                            cnt_tile, [dst16],
                            jnp.ones((16,), jnp.float32))
            for h in sh:
                h.wait()

        plsc.subcore_barrier()

        for r in range(n_lin):
            pltpu.async_copy(agg_sh.at[lin_v.at[r]], rows_v.at[0],
                             sem).wait()
            pltpu.sync_copy(
                rows_v.at[0],
                agg_hbm.at[pl.ds(cid * n_pad + base + r * CHUNK, CHUNK)])
        if with_cnt:
            pltpu.sync_copy(cnt_tile, cnt_hbm.at[wid])

    lin = jnp.arange(n_pad, dtype=jnp.int32).reshape(NS, n_lin, CHUNK)
    res = k(t, idx3, lin)
    if with_cnt:
        agg, cnt = res
        return agg.reshape(NC, n_pad, d), cnt
    if isinstance(res, (list, tuple)):
        res = res[0]
    return res.reshape(NC, n_pad, d)


def _tc_pre(x, w_l, w_r):
    n, d_in = x.shape
    d_out = w_l.shape[1]
    grid = n // ROW_BLOCK

    def body(x_ref, wl_ref, wr_ref, t_ref, r_ref):
        xb = x_ref[...]
        t_ref[...] = jnp.dot(xb, wl_ref[...],
                             preferred_element_type=jnp.float32)
        r_ref[...] = jnp.dot(xb, wr_ref[...],
                             preferred_element_type=jnp.float32)

    return pl.pallas_call(
        body,
        grid=(grid,),
        in_specs=[
            pl.BlockSpec((ROW_BLOCK, d_in), lambda i: (i, 0)),
            pl.BlockSpec((d_in, d_out), lambda i: (0, 0)),
            pl.BlockSpec((d_in, d_out), lambda i: (0, 0)),
        ],
        out_specs=[
            pl.BlockSpec((ROW_BLOCK, d_out), lambda i: (i, 0)),
            pl.BlockSpec((ROW_BLOCK, d_out), lambda i: (i, 0)),
        ],
        out_shape=[
            jax.ShapeDtypeStruct((n, d_out), jnp.float32),
            jax.ShapeDtypeStruct((n, d_out), jnp.float32),
        ],
    )(x, w_l, w_r)


def _tc_mid(agg, cnt, r1, b1, w_r2):
    n, d_h = r1.shape
    d2 = w_r2.shape[1]
    grid = n // ROW_BLOCK

    def body(agg_ref, cnt_ref, r1_ref, b1_ref, wr_ref, h_ref, r2_ref):
        s = jnp.sum(agg_ref[...], axis=0)
        c = jnp.sum(cnt_ref[...], axis=1)[:, None]
        mean = s / jnp.maximum(c, 1.0)
        h = jnp.maximum(mean + b1_ref[...] + r1_ref[...], 0.0)
        h_ref[...] = h
        r2_ref[...] = jnp.dot(h, wr_ref[...],
                              preferred_element_type=jnp.float32)

    return pl.pallas_call(
        body,
        grid=(grid,),
        in_specs=[
            pl.BlockSpec((NC, ROW_BLOCK, d_h), lambda i: (0, i, 0)),
            pl.BlockSpec((ROW_BLOCK, NW), lambda i: (i, 0)),
            pl.BlockSpec((ROW_BLOCK, d_h), lambda i: (i, 0)),
            pl.BlockSpec((1, d_h), lambda i: (0, 0)),
            pl.BlockSpec((d_h, d2), lambda i: (0, 0)),
        ],
        out_specs=[
            pl.BlockSpec((ROW_BLOCK, d_h), lambda i: (i, 0)),
            pl.BlockSpec((ROW_BLOCK, d2), lambda i: (i, 0)),
        ],
        out_shape=[
            jax.ShapeDtypeStruct((n, d_h), jnp.float32),
            jax.ShapeDtypeStruct((n, d2), jnp.float32),
        ],
    )(agg, cnt, r1, b1, w_r2)


def _tc_post(agg, cnt, r2, b2, w_l2):
    n, d2 = r2.shape
    d_h = w_l2.shape[0]
    grid = n // ROW_BLOCK

    def body(agg_ref, cnt_ref, r2_ref, b2_ref, wl_ref, o_ref):
        s = jnp.sum(agg_ref[...], axis=0)
        c = jnp.sum(cnt_ref[...], axis=1)[:, None]
        mean = s / jnp.maximum(c, 1.0)
        m2 = jnp.dot(mean, wl_ref[...], preferred_element_type=jnp.float32)
        o_ref[...] = jnp.maximum(m2 + b2_ref[...] + r2_ref[...], 0.0)

    return pl.pallas_call(
        body,
        grid=(grid,),
        in_specs=[
            pl.BlockSpec((NC, ROW_BLOCK, d_h), lambda i: (0, i, 0)),
            pl.BlockSpec((ROW_BLOCK, NW), lambda i: (i, 0)),
            pl.BlockSpec((ROW_BLOCK, d2), lambda i: (i, 0)),
            pl.BlockSpec((1, d2), lambda i: (0, 0)),
            pl.BlockSpec((d_h, d2), lambda i: (0, 0)),
        ],
        out_specs=pl.BlockSpec((ROW_BLOCK, d2), lambda i: (i, 0)),
        out_shape=jax.ShapeDtypeStruct((n, d2), jnp.float32),
    )(agg, cnt, r2, b2, w_l2)


def kernel(x, edge_index, W_l1, b_l1, W_r1, W_l2, b_l2, W_r2):
    n = x.shape[0]
    e = edge_index.shape[1]
    n_pad = ((n + NS * CHUNK) // (NS * CHUNK)) * (NS * CHUNK)
    grain = NW * CHUNK * KGROUP
    n_chunks = (-(-e // grain)) * KGROUP
    e_pad = NW * n_chunks * CHUNK

    src = edge_index[0]
    dst = edge_index[1]
    if e_pad > e:
        pad = e_pad - e
        src = jnp.concatenate([src, jnp.zeros((pad,), jnp.int32)])
        dst = jnp.concatenate([dst, jnp.full((pad,), n, jnp.int32)])
    idx3 = jnp.stack([src.reshape(NW, n_chunks, CHUNK),
                      dst.reshape(NW, n_chunks, CHUNK)], axis=2)
    idx3 = idx3.reshape(NW * 2 * n_chunks, CHUNK)

    b1 = b_l1.reshape(1, -1)
    b2 = b_l2.reshape(1, -1)

    t1, r1 = _tc_pre(x, W_l1, W_r1)
    agg1, cnt = _sc_segment_sum(t1, idx3, n_pad, with_cnt=True)
    cnt = cnt.T
    h1, r2 = _tc_mid(agg1, cnt, r1, b1, W_r2)

    agg2 = _sc_segment_sum(h1, idx3, n_pad, with_cnt=False)
    return _tc_post(agg2, cnt, r2, b2, W_l2)


# speedup vs baseline: 1.1419x; 1.1419x over previous
import dataclasses
import functools

import jax
import jax.numpy as jnp
from jax import lax
from jax.experimental import pallas as pl
from jax.experimental.pallas import tpu as pltpu
from jax.experimental.pallas import tpu_sc as plsc

NC = 2
NS = 16
NW = NC * NS
CHUNK = 32
KGROUP = 8
ROW_BLOCK = 2000


def _sc_segment_sum(t, idx3, n_pad, with_cnt):
    n_chunks = idx3.shape[0] // (2 * NW)
    d = t.shape[1]
    rows_per_sub = n_pad // NS
    n_groups = n_chunks // KGROUP
    n_lin = rows_per_sub // CHUNK

    out_type = [jax.ShapeDtypeStruct((NC * n_pad, d), jnp.float32)]
    scratch = [
        pltpu.VMEM((2 * KGROUP, CHUNK), jnp.int32),
        pltpu.VMEM((n_lin, CHUNK), jnp.int32),
        pltpu.VMEM((KGROUP, CHUNK, d), jnp.float32),
        pltpu.SemaphoreType.DMA,
        pltpu.SemaphoreType.DMA,
        pltpu.VMEM_SHARED((n_pad, d), jnp.float32),
    ]
    if with_cnt:
        out_type.append(jax.ShapeDtypeStruct((NW, n_pad), jnp.float32))
        scratch.append(pltpu.VMEM((n_pad,), jnp.float32))

    mesh = plsc.VectorSubcoreMesh(core_axis_name="c", subcore_axis_name="s",
                                  num_cores=NC)
    cp = pltpu.CompilerParams()
    if "needs_layout_passes" in pltpu.CompilerParams.__dataclass_fields__:
        cp = dataclasses.replace(cp, needs_layout_passes=False)

    @functools.partial(pl.kernel, out_type=out_type, mesh=mesh,
                       scratch_types=scratch, compiler_params=cp)
    def k(t_hbm, idx_hbm, lin_hbm, *rest):
        if with_cnt:
            (agg_hbm, cnt_hbm, idx_v, lin_v, rows_v, gsem, ssem,
             agg_sh, cnt_tile) = rest
        else:
            agg_hbm, idx_v, lin_v, rows_v, gsem, ssem, agg_sh = rest
        sem = gsem
        cid = lax.axis_index("c")
        sid = lax.axis_index("s")
        wid = cid * NS + sid
        base = sid * rows_per_sub
        idx_base = wid * 2 * n_chunks

        pltpu.sync_copy(lin_hbm.at[sid], lin_v)

        @pl.loop(0, CHUNK)
        def _(i):
            @pl.loop(0, d, step=16)
            def _(j):
                rows_v[0, i, pl.ds(j, 16)] = jnp.zeros((16,), jnp.float32)

        if with_cnt:
            @pl.loop(0, n_pad, step=16)
            def _(i):
                cnt_tile[pl.ds(i, 16)] = jnp.zeros((16,), jnp.float32)

        for r in range(n_lin):
            pltpu.sync_copy(rows_v.at[0], agg_sh.at[lin_v.at[r]])

        plsc.subcore_barrier()

        @pl.loop(0, n_groups)
        def _(g):
            pltpu.sync_copy(
                idx_hbm.at[pl.ds(idx_base + g * 2 * KGROUP, 2 * KGROUP)],
                idx_v)
            gh = [pltpu.async_copy(t_hbm.at[idx_v.at[2 * kk]],
                                   rows_v.at[kk], gsem)
                  for kk in range(KGROUP)]
            for h in gh:
                h.wait()
            sh = [pltpu.async_copy(rows_v.at[kk],
                                   agg_sh.at[idx_v.at[2 * kk + 1]], ssem,
                                   add=True)
                  for kk in range(KGROUP)]
            if with_cnt:
                for kk in range(KGROUP):
                    for l in range(CHUNK // 16):
                        dst16 = idx_v[2 * kk + 1, pl.ds(l * 16, 16)]
                        plsc.addupdate_scatter(
                            cnt_tile, [dst16],
                            jnp.ones((16,), jnp.float32))
            for h in sh:
                h.wait()

        plsc.subcore_barrier()

        for r in range(n_lin):
            pltpu.async_copy(agg_sh.at[lin_v.at[r]], rows_v.at[0],
                             sem).wait()
            pltpu.sync_copy(
                rows_v.at[0],
                agg_hbm.at[pl.ds(cid * n_pad + base + r * CHUNK, CHUNK)])
        if with_cnt:
            pltpu.sync_copy(cnt_tile, cnt_hbm.at[wid])

    lin = jnp.arange(n_pad, dtype=jnp.int32).reshape(NS, n_lin, CHUNK)
    res = k(t, idx3, lin)
    if with_cnt:
        agg, cnt = res
        return agg.reshape(NC, n_pad, d), cnt
    if isinstance(res, (list, tuple)):
        res = res[0]
    return res.reshape(NC, n_pad, d)


def _tc_pre(x, w_l, w_r):
    n, d_in = x.shape
    d_out = w_l.shape[1]
    grid = n // ROW_BLOCK

    def body(x_ref, wl_ref, wr_ref, t_ref, r_ref):
        xb = x_ref[...]
        t_ref[...] = jnp.dot(xb, wl_ref[...],
                             preferred_element_type=jnp.float32)
        r_ref[...] = jnp.dot(xb, wr_ref[...],
                             preferred_element_type=jnp.float32)

    return pl.pallas_call(
        body,
        grid=(grid,),
        in_specs=[
            pl.BlockSpec((ROW_BLOCK, d_in), lambda i: (i, 0)),
            pl.BlockSpec((d_in, d_out), lambda i: (0, 0)),
            pl.BlockSpec((d_in, d_out), lambda i: (0, 0)),
        ],
        out_specs=[
            pl.BlockSpec((ROW_BLOCK, d_out), lambda i: (i, 0)),
            pl.BlockSpec((ROW_BLOCK, d_out), lambda i: (i, 0)),
        ],
        out_shape=[
            jax.ShapeDtypeStruct((n, d_out), jnp.float32),
            jax.ShapeDtypeStruct((n, d_out), jnp.float32),
        ],
    )(x, w_l, w_r)


def _tc_mid(agg, cnt, r1, b1, w_r2):
    n, d_h = r1.shape
    d2 = w_r2.shape[1]
    grid = n // ROW_BLOCK

    def body(agg_ref, cnt_ref, r1_ref, b1_ref, wr_ref, h_ref, r2_ref):
        s = jnp.sum(agg_ref[...], axis=0)
        c = jnp.sum(cnt_ref[...], axis=1)[:, None]
        mean = s / jnp.maximum(c, 1.0)
        h = jnp.maximum(mean + b1_ref[...] + r1_ref[...], 0.0)
        h_ref[...] = h
        r2_ref[...] = jnp.dot(h, wr_ref[...],
                              preferred_element_type=jnp.float32)

    return pl.pallas_call(
        body,
        grid=(grid,),
        in_specs=[
            pl.BlockSpec((NC, ROW_BLOCK, d_h), lambda i: (0, i, 0)),
            pl.BlockSpec((ROW_BLOCK, NW), lambda i: (i, 0)),
            pl.BlockSpec((ROW_BLOCK, d_h), lambda i: (i, 0)),
            pl.BlockSpec((1, d_h), lambda i: (0, 0)),
            pl.BlockSpec((d_h, d2), lambda i: (0, 0)),
        ],
        out_specs=[
            pl.BlockSpec((ROW_BLOCK, d_h), lambda i: (i, 0)),
            pl.BlockSpec((ROW_BLOCK, d2), lambda i: (i, 0)),
        ],
        out_shape=[
            jax.ShapeDtypeStruct((n, d_h), jnp.float32),
            jax.ShapeDtypeStruct((n, d2), jnp.float32),
        ],
    )(agg, cnt, r1, b1, w_r2)


def _tc_post(agg, cnt, r2, b2, w_l2):
    n, d2 = r2.shape
    d_h = w_l2.shape[0]
    grid = n // ROW_BLOCK

    def body(agg_ref, cnt_ref, r2_ref, b2_ref, wl_ref, o_ref):
        s = jnp.sum(agg_ref[...], axis=0)
        c = jnp.sum(cnt_ref[...], axis=1)[:, None]
        mean = s / jnp.maximum(c, 1.0)
        m2 = jnp.dot(mean, wl_ref[...], preferred_element_type=jnp.float32)
        o_ref[...] = jnp.maximum(m2 + b2_ref[...] + r2_ref[...], 0.0)

    return pl.pallas_call(
        body,
        grid=(grid,),
        in_specs=[
            pl.BlockSpec((NC, ROW_BLOCK, d_h), lambda i: (0, i, 0)),
            pl.BlockSpec((ROW_BLOCK, NW), lambda i: (i, 0)),
            pl.BlockSpec((ROW_BLOCK, d2), lambda i: (i, 0)),
            pl.BlockSpec((1, d2), lambda i: (0, 0)),
            pl.BlockSpec((d_h, d2), lambda i: (0, 0)),
        ],
        out_specs=pl.BlockSpec((ROW_BLOCK, d2), lambda i: (i, 0)),
        out_shape=jax.ShapeDtypeStruct((n, d2), jnp.float32),
    )(agg, cnt, r2, b2, w_l2)


def kernel(x, edge_index, W_l1, b_l1, W_r1, W_l2, b_l2, W_r2):
    n = x.shape[0]
    e = edge_index.shape[1]
    n_pad = ((n + NS * CHUNK) // (NS * CHUNK)) * (NS * CHUNK)
    grain = NW * CHUNK * KGROUP
    n_chunks = (-(-e // grain)) * KGROUP
    e_pad = NW * n_chunks * CHUNK

    src = edge_index[0]
    dst = edge_index[1]
    if e_pad > e:
        pad = e_pad - e
        src = jnp.concatenate([src, jnp.zeros((pad,), jnp.int32)])
        dst = jnp.concatenate(
            [dst, n + (jnp.arange(pad, dtype=jnp.int32) % (n_pad - n))])
    idx3 = jnp.stack([src.reshape(NW, n_chunks, CHUNK),
                      dst.reshape(NW, n_chunks, CHUNK)], axis=2)
    idx3 = idx3.reshape(NW * 2 * n_chunks, CHUNK)

    b1 = b_l1.reshape(1, -1)
    b2 = b_l2.reshape(1, -1)

    t1, r1 = _tc_pre(x, W_l1, W_r1)
    agg1, cnt = _sc_segment_sum(t1, idx3, n_pad, with_cnt=True)
    cnt = cnt.T
    h1, r2 = _tc_mid(agg1, cnt, r1, b1, W_r2)

    agg2 = _sc_segment_sum(h1, idx3, n_pad, with_cnt=False)
    return _tc_post(agg2, cnt, r2, b2, W_l2)


# speedup vs baseline: 1.1914x; 1.0434x over previous
import dataclasses
import functools

import jax
import jax.numpy as jnp
from jax import lax
from jax.experimental import pallas as pl
from jax.experimental.pallas import tpu as pltpu
from jax.experimental.pallas import tpu_sc as plsc

NC = 2
NS = 16
NW = NC * NS
CHUNK = 64
KGROUP = 4
ROW_BLOCK = 2000


def _sc_segment_sum(t, idx3, n_pad, with_cnt):
    n_chunks = idx3.shape[0] // (2 * NW)
    d = t.shape[1]
    rows_per_sub = n_pad // NS
    n_groups = n_chunks // KGROUP
    n_lin = rows_per_sub // CHUNK

    out_type = [jax.ShapeDtypeStruct((NC * n_pad, d), jnp.float32)]
    scratch = [
        pltpu.VMEM((2 * KGROUP, CHUNK), jnp.int32),
        pltpu.VMEM((n_lin, CHUNK), jnp.int32),
        pltpu.VMEM((KGROUP, CHUNK, d), jnp.float32),
        pltpu.SemaphoreType.DMA,
        pltpu.SemaphoreType.DMA,
        pltpu.VMEM_SHARED((n_pad, d), jnp.float32),
    ]
    if with_cnt:
        out_type.append(jax.ShapeDtypeStruct((NW, n_pad), jnp.float32))
        scratch.append(pltpu.VMEM((n_pad,), jnp.float32))

    mesh = plsc.VectorSubcoreMesh(core_axis_name="c", subcore_axis_name="s",
                                  num_cores=NC)
    cp = pltpu.CompilerParams()
    if "needs_layout_passes" in pltpu.CompilerParams.__dataclass_fields__:
        cp = dataclasses.replace(cp, needs_layout_passes=False)

    @functools.partial(pl.kernel, out_type=out_type, mesh=mesh,
                       scratch_types=scratch, compiler_params=cp)
    def k(t_hbm, idx_hbm, lin_hbm, *rest):
        if with_cnt:
            (agg_hbm, cnt_hbm, idx_v, lin_v, rows_v, gsem, ssem,
             agg_sh, cnt_tile) = rest
        else:
            agg_hbm, idx_v, lin_v, rows_v, gsem, ssem, agg_sh = rest
        sem = gsem
        cid = lax.axis_index("c")
        sid = lax.axis_index("s")
        wid = cid * NS + sid
        base = sid * rows_per_sub
        idx_base = wid * 2 * n_chunks

        pltpu.sync_copy(lin_hbm.at[sid], lin_v)

        @pl.loop(0, CHUNK)
        def _(i):
            @pl.loop(0, d, step=16)
            def _(j):
                rows_v[0, i, pl.ds(j, 16)] = jnp.zeros((16,), jnp.float32)

        if with_cnt:
            @pl.loop(0, n_pad, step=16)
            def _(i):
                cnt_tile[pl.ds(i, 16)] = jnp.zeros((16,), jnp.float32)

        for r in range(n_lin):
            pltpu.sync_copy(rows_v.at[0], agg_sh.at[lin_v.at[r]])

        plsc.subcore_barrier()

        @pl.loop(0, n_groups)
        def _(g):
            pltpu.sync_copy(
                idx_hbm.at[pl.ds(idx_base + g * 2 * KGROUP, 2 * KGROUP)],
                idx_v)
            gh = [pltpu.async_copy(t_hbm.at[idx_v.at[2 * kk]],
                                   rows_v.at[kk], gsem)
                  for kk in range(KGROUP)]
            for h in gh:
                h.wait()
            sh = [pltpu.async_copy(rows_v.at[kk],
                                   agg_sh.at[idx_v.at[2 * kk + 1]], ssem,
                                   add=True)
                  for kk in range(KGROUP)]
            if with_cnt:
                for kk in range(KGROUP):
                    for l in range(CHUNK // 16):
                        dst16 = idx_v[2 * kk + 1, pl.ds(l * 16, 16)]
                        plsc.addupdate_scatter(
                            cnt_tile, [dst16],
                            jnp.ones((16,), jnp.float32))
            for h in sh:
                h.wait()

        plsc.subcore_barrier()

        for r in range(n_lin):
            pltpu.async_copy(agg_sh.at[lin_v.at[r]], rows_v.at[0],
                             sem).wait()
            pltpu.sync_copy(
                rows_v.at[0],
                agg_hbm.at[pl.ds(cid * n_pad + base + r * CHUNK, CHUNK)])
        if with_cnt:
            pltpu.sync_copy(cnt_tile, cnt_hbm.at[wid])

    lin = jnp.arange(n_pad, dtype=jnp.int32).reshape(NS, n_lin, CHUNK)
    res = k(t, idx3, lin)
    if with_cnt:
        agg, cnt = res
        return agg.reshape(NC, n_pad, d), cnt
    if isinstance(res, (list, tuple)):
        res = res[0]
    return res.reshape(NC, n_pad, d)


def _tc_pre(x, w_l, w_r):
    n, d_in = x.shape
    d_out = w_l.shape[1]
    grid = n // ROW_BLOCK

    def body(x_ref, wl_ref, wr_ref, t_ref, r_ref):
        xb = x_ref[...]
        t_ref[...] = jnp.dot(xb, wl_ref[...],
                             preferred_element_type=jnp.float32)
        r_ref[...] = jnp.dot(xb, wr_ref[...],
                             preferred_element_type=jnp.float32)

    return pl.pallas_call(
        body,
        grid=(grid,),
        in_specs=[
            pl.BlockSpec((ROW_BLOCK, d_in), lambda i: (i, 0)),
            pl.BlockSpec((d_in, d_out), lambda i: (0, 0)),
            pl.BlockSpec((d_in, d_out), lambda i: (0, 0)),
        ],
        out_specs=[
            pl.BlockSpec((ROW_BLOCK, d_out), lambda i: (i, 0)),
            pl.BlockSpec((ROW_BLOCK, d_out), lambda i: (i, 0)),
        ],
        out_shape=[
            jax.ShapeDtypeStruct((n, d_out), jnp.float32),
            jax.ShapeDtypeStruct((n, d_out), jnp.float32),
        ],
    )(x, w_l, w_r)


def _tc_mid(agg, cnt, r1, b1, w_r2):
    n, d_h = r1.shape
    d2 = w_r2.shape[1]
    grid = n // ROW_BLOCK

    def body(agg_ref, cnt_ref, r1_ref, b1_ref, wr_ref, h_ref, r2_ref):
        s = jnp.sum(agg_ref[...], axis=0)
        c = jnp.sum(cnt_ref[...], axis=1)[:, None]
        mean = s / jnp.maximum(c, 1.0)
        h = jnp.maximum(mean + b1_ref[...] + r1_ref[...], 0.0)
        h_ref[...] = h
        r2_ref[...] = jnp.dot(h, wr_ref[...],
                              preferred_element_type=jnp.float32)

    return pl.pallas_call(
        body,
        grid=(grid,),
        in_specs=[
            pl.BlockSpec((NC, ROW_BLOCK, d_h), lambda i: (0, i, 0)),
            pl.BlockSpec((ROW_BLOCK, NW), lambda i: (i, 0)),
            pl.BlockSpec((ROW_BLOCK, d_h), lambda i: (i, 0)),
            pl.BlockSpec((1, d_h), lambda i: (0, 0)),
            pl.BlockSpec((d_h, d2), lambda i: (0, 0)),
        ],
        out_specs=[
            pl.BlockSpec((ROW_BLOCK, d_h), lambda i: (i, 0)),
            pl.BlockSpec((ROW_BLOCK, d2), lambda i: (i, 0)),
        ],
        out_shape=[
            jax.ShapeDtypeStruct((n, d_h), jnp.float32),
            jax.ShapeDtypeStruct((n, d2), jnp.float32),
        ],
    )(agg, cnt, r1, b1, w_r2)


def _tc_post(agg, cnt, r2, b2, w_l2):
    n, d2 = r2.shape
    d_h = w_l2.shape[0]
    grid = n // ROW_BLOCK

    def body(agg_ref, cnt_ref, r2_ref, b2_ref, wl_ref, o_ref):
        s = jnp.sum(agg_ref[...], axis=0)
        c = jnp.sum(cnt_ref[...], axis=1)[:, None]
        mean = s / jnp.maximum(c, 1.0)
        m2 = jnp.dot(mean, wl_ref[...], preferred_element_type=jnp.float32)
        o_ref[...] = jnp.maximum(m2 + b2_ref[...] + r2_ref[...], 0.0)

    return pl.pallas_call(
        body,
        grid=(grid,),
        in_specs=[
            pl.BlockSpec((NC, ROW_BLOCK, d_h), lambda i: (0, i, 0)),
            pl.BlockSpec((ROW_BLOCK, NW), lambda i: (i, 0)),
            pl.BlockSpec((ROW_BLOCK, d2), lambda i: (i, 0)),
            pl.BlockSpec((1, d2), lambda i: (0, 0)),
            pl.BlockSpec((d_h, d2), lambda i: (0, 0)),
        ],
        out_specs=pl.BlockSpec((ROW_BLOCK, d2), lambda i: (i, 0)),
        out_shape=jax.ShapeDtypeStruct((n, d2), jnp.float32),
    )(agg, cnt, r2, b2, w_l2)


def kernel(x, edge_index, W_l1, b_l1, W_r1, W_l2, b_l2, W_r2):
    n = x.shape[0]
    e = edge_index.shape[1]
    n_pad = ((n + NS * CHUNK) // (NS * CHUNK)) * (NS * CHUNK)
    grain = NW * CHUNK * KGROUP
    n_chunks = (-(-e // grain)) * KGROUP
    e_pad = NW * n_chunks * CHUNK

    src = edge_index[0]
    dst = edge_index[1]
    if e_pad > e:
        pad = e_pad - e
        src = jnp.concatenate([src, jnp.zeros((pad,), jnp.int32)])
        dst = jnp.concatenate(
            [dst, n + (jnp.arange(pad, dtype=jnp.int32) % (n_pad - n))])
    idx3 = jnp.stack([src.reshape(NW, n_chunks, CHUNK),
                      dst.reshape(NW, n_chunks, CHUNK)], axis=2)
    idx3 = idx3.reshape(NW * 2 * n_chunks, CHUNK)

    b1 = b_l1.reshape(1, -1)
    b2 = b_l2.reshape(1, -1)

    t1, r1 = _tc_pre(x, W_l1, W_r1)
    agg1, cnt = _sc_segment_sum(t1, idx3, n_pad, with_cnt=True)
    cnt = cnt.T
    h1, r2 = _tc_mid(agg1, cnt, r1, b1, W_r2)

    agg2 = _sc_segment_sum(h1, idx3, n_pad, with_cnt=False)
    return _tc_post(agg2, cnt, r2, b2, W_l2)
